# Initial kernel scaffold; baseline (speedup 1.0000x reference)
import jax
import jax.numpy as jnp
from jax.experimental import pallas as pl


def kernel(atomic_numbers, edge_attr, edge_index, batch, W_embed, R1, R2, R3, R4, W_up, W_sc, W_species, Wp1, Wp2, Wp3, W_readout, b_readout):
    raise NotImplementedError("write your pallas kernel here")


# speedup vs baseline: 92.6884x; 92.6884x over previous
import functools

import jax
import jax.numpy as jnp
from jax import lax
from jax.experimental import pallas as pl
from jax.experimental.pallas import tpu as pltpu
from jax.experimental.pallas import tpu_sc as plsc

_HID = 128
_NB = 2000
_EB = 2000
_CH = 128
_NC, _NS = 2, 16
_NW = _NC * _NS
_NG = 64
_AVG = 35.0
_RMAX = 5.0


def _silu(x):
    return x * (1.0 / (1.0 + jnp.exp(-x)))


def _dot(a, b):
    return jnp.dot(a, b, preferred_element_type=jnp.float32)


def _dotT(aT, b):
    return lax.dot_general(aT, b, (((0,), (0,)), ((), ())),
                           preferred_element_type=jnp.float32)


def _edge_mlp_body(ea_ref, r1_ref, r2_ref, r3_ref, r4_ref, w_ref):
    ea = ea_ref[...]
    r = jnp.sqrt(jnp.sum(ea * ea, axis=1, keepdims=True))
    nfreq = lax.broadcasted_iota(jnp.int32, (1, 8), 1).astype(jnp.float32) + 1.0
    pref = jnp.sqrt(2.0 / _RMAX)
    bess = pref * jnp.sin(nfreq * (jnp.pi / _RMAX) * r) / jnp.maximum(r, 1e-6)
    u = r * (1.0 / _RMAX)
    u2 = u * u
    u5 = u2 * u2 * u
    env = 1.0 - 21.0 * u5 + 35.0 * u5 * u - 15.0 * u5 * u2
    env = jnp.where(u < 1.0, env, 0.0)
    ef = bess * env
    h = _silu(_dot(ef, r1_ref[...]))
    h = _silu(_dot(h, r2_ref[...]))
    h = _silu(_dot(h, r3_ref[...]))
    w_ref[...] = _dot(h, r4_ref[...])


def _edge_mlp(edge_attr, r1, r2, r3, r4):
    e = edge_attr.shape[0]
    grid = (e // _EB,)
    return pl.pallas_call(
        _edge_mlp_body,
        grid=grid,
        in_specs=[
            pl.BlockSpec((_EB, 3), lambda i: (i, 0)),
            pl.BlockSpec(r1.shape, lambda i: (0, 0)),
            pl.BlockSpec(r2.shape, lambda i: (0, 0)),
            pl.BlockSpec(r3.shape, lambda i: (0, 0)),
            pl.BlockSpec(r4.shape, lambda i: (0, 0)),
        ],
        out_specs=pl.BlockSpec((_EB, _HID), lambda i: (i, 0)),
        out_shape=jax.ShapeDtypeStruct((e, _HID), jnp.float32),
    )(edge_attr, r1, r2, r3, r4)


def _node_prep_body(an_ref, we_ref, wu_ref, ws0_ref, ws1_ref,
                    f0up_ref, s0_ref, s1_ref):
    an = an_ref[0]
    ids = lax.broadcasted_iota(jnp.int32, (we_ref.shape[0], 1), 0)
    oht = (ids == an).astype(jnp.float32)
    z0 = _dotT(oht, we_ref[...])
    f0up_ref[...] = _dot(z0, wu_ref[...])
    s0_ref[...] = _dotT(oht, ws0_ref[...])
    s1_ref[...] = _dotT(oht, ws1_ref[...])


def _node_prep(an3, w_embed, w_up0, ws0, ws1):
    nblk = an3.shape[0]
    n = nblk * _NB
    out = jax.ShapeDtypeStruct((n, _HID), jnp.float32)
    return pl.pallas_call(
        _node_prep_body,
        grid=(nblk,),
        in_specs=[
            pl.BlockSpec((1, 1, _NB), lambda i: (i, 0, 0)),
            pl.BlockSpec(w_embed.shape, lambda i: (0, 0)),
            pl.BlockSpec(w_up0.shape, lambda i: (0, 0)),
            pl.BlockSpec(ws0.shape, lambda i: (0, 0)),
            pl.BlockSpec(ws1.shape, lambda i: (0, 0)),
        ],
        out_specs=[pl.BlockSpec((_NB, _HID), lambda i: (i, 0))] * 3,
        out_shape=[out] * 3,
    )(an3, w_embed, w_up0, ws0, ws1)


def _sc_msg(f_nodes, w_edges, src2, dst2):
    n, d = f_nodes.shape
    n_chunks = src2.shape[0]
    per_w = n_chunks // _NW
    extra = n_chunks - per_w * _NW
    rbase = 624
    rlast = n - rbase * (_NS - 1)
    mesh = plsc.VectorSubcoreMesh(core_axis_name="c", subcore_axis_name="s")
    out_t = [jax.ShapeDtypeStruct((n, d), jnp.float32)] * 2

    @functools.partial(
        pl.kernel, out_type=out_t, mesh=mesh,
        scratch_types=[
            pltpu.VMEM((1, _CH), jnp.int32),
            pltpu.VMEM((1, _CH), jnp.int32),
            pltpu.VMEM((_CH, d), jnp.float32),
            pltpu.VMEM((_CH, d), jnp.float32),
            pltpu.VMEM_SHARED((n, d), jnp.float32),
            pltpu.SemaphoreType.DMA,
        ])
    def k(f_hbm, w_hbm, src_hbm, dst_hbm, out0, out1,
          src_v, dst_v, se_v, w_v, acc, sem):
        cid = lax.axis_index("c")
        sid = lax.axis_index("s")
        wid = cid * _NS + sid

        base = sid * rbase

        @pl.loop(0, _CH)
        def _z(rr):
            for cc in range(d // 16):
                w_v[rr, pl.ds(cc * 16, 16)] = jnp.zeros((16,), jnp.float32)

        @pl.loop(0, rbase // _CH)
        def _zc(j):
            pltpu.sync_copy(w_v, acc.at[pl.ds(base + j * _CH, _CH)])

        @pl.when(sid < _NS - 1)
        def _zt():
            pltpu.sync_copy(w_v.at[pl.ds(0, rbase % _CH)],
                            acc.at[pl.ds(base + rbase - rbase % _CH,
                                         rbase % _CH)])

        @pl.when(sid == _NS - 1)
        def _zt2():
            pltpu.sync_copy(w_v, acc.at[pl.ds(base + (rbase // _CH) * _CH,
                                              rlast - (rbase // _CH) * _CH)])

        plsc.subcore_barrier()

        nb = per_w + jnp.where(wid < extra, 1, 0)

        @pl.loop(0, per_w + 1)
        def _c(c):
            @pl.when(c < nb)
            def _():
                g = wid + c * _NW
                pltpu.sync_copy(src_hbm.at[pl.ds(g, 1)], src_v)
                pltpu.sync_copy(dst_hbm.at[pl.ds(g, 1)], dst_v)
                pltpu.async_copy(f_hbm.at[src_v.at[0]], se_v, sem).wait()
                pltpu.sync_copy(w_hbm.at[pl.ds(g * _CH, _CH)], w_v)

                @pl.loop(0, _CH)
                def _m(rr):
                    for cc in range(d // 16):
                        sl = pl.ds(cc * 16, 16)
                        w_v[rr, sl] = w_v[rr, sl] * se_v[rr, sl]

                pltpu.sync_copy(w_v, acc.at[dst_v.at[0]], add=True)

        plsc.subcore_barrier()

        @pl.when(cid == 0)
        def _o0():
            @pl.when(sid < _NS - 1)
            def _():
                pltpu.sync_copy(acc.at[pl.ds(base, rbase)],
                                out0.at[pl.ds(base, rbase)])

            @pl.when(sid == _NS - 1)
            def _():
                pltpu.sync_copy(acc.at[pl.ds(base, rlast)],
                                out0.at[pl.ds(base, rlast)])

        @pl.when(cid == 1)
        def _o1():
            @pl.when(sid < _NS - 1)
            def _():
                pltpu.sync_copy(acc.at[pl.ds(base, rbase)],
                                out1.at[pl.ds(base, rbase)])

            @pl.when(sid == _NS - 1)
            def _():
                pltpu.sync_copy(acc.at[pl.ds(base, rlast)],
                                out1.at[pl.ds(base, rlast)])

    return k(f_nodes, w_edges, src2, dst2)


def _l0_body(ma_ref, mb_ref, s0_ref, wp1_ref, wp2_ref, wp3_ref, wu1_ref,
             f1_ref, f1up_ref):
    msg = (ma_ref[...] + mb_ref[...]) * (1.0 / _AVG)
    msg2 = msg * msg
    o = _dot(msg, wp1_ref[...])
    o = o + _dot(msg2, wp2_ref[...])
    o = o + _dot(msg2 * msg, wp3_ref[...])
    f1 = o * s0_ref[...]
    f1_ref[...] = f1
    f1up_ref[...] = _dot(f1, wu1_ref[...])


def _layer0_update(ma, mb, s0, wp1, wp2, wp3, wu1):
    n = ma.shape[0]
    out = jax.ShapeDtypeStruct((n, _HID), jnp.float32)
    full = pl.BlockSpec((_HID, _HID), lambda i: (0, 0))
    blk = pl.BlockSpec((_NB, _HID), lambda i: (i, 0))
    return pl.pallas_call(
        _l0_body,
        grid=(n // _NB,),
        in_specs=[blk, blk, blk, full, full, full, full],
        out_specs=[blk, blk],
        out_shape=[out, out],
    )(ma, mb, s0, wp1, wp2, wp3, wu1)


def _final_body(ma_ref, mb_ref, s1_ref, f1_ref, bt_ref, wp1_ref, wp2_ref,
                wp3_ref, wsc_ref, wr_ref, br_ref, sums_ref, cnt_ref):
    i = pl.program_id(0)
    msg = (ma_ref[...] + mb_ref[...]) * (1.0 / _AVG)
    msg2 = msg * msg
    o = _dot(msg, wp1_ref[...])
    o = o + _dot(msg2, wp2_ref[...])
    o = o + _dot(msg2 * msg, wp3_ref[...])
    o = o * s1_ref[...]
    o = o + _dot(f1_ref[...], wsc_ref[...])
    nout = _dot(o, wr_ref[...]) + br_ref[...]
    bt = bt_ref[0]
    gids = lax.broadcasted_iota(jnp.int32, (_NG, 1), 0)
    oht = (gids == bt).astype(jnp.float32)
    s = _dot(oht, nout)
    c = jnp.sum(oht, axis=1, keepdims=True)

    @pl.when(i == 0)
    def _():
        sums_ref[...] = jnp.zeros_like(sums_ref)
        cnt_ref[...] = jnp.zeros_like(cnt_ref)

    sums_ref[...] += s
    cnt_ref[...] += c


def _final(ma, mb, s1, f1, bt3, wp1, wp2, wp3, wsc, wr, br):
    n = ma.shape[0]
    nout = wr.shape[1]
    blk = pl.BlockSpec((_NB, _HID), lambda i: (i, 0))
    full = pl.BlockSpec((_HID, _HID), lambda i: (0, 0))
    return pl.pallas_call(
        _final_body,
        grid=(n // _NB,),
        in_specs=[blk, blk, blk, blk,
                  pl.BlockSpec((1, 1, _NB), lambda i: (i, 0, 0)),
                  full, full, full, full,
                  pl.BlockSpec(wr.shape, lambda i: (0, 0)),
                  pl.BlockSpec(br.shape, lambda i: (0, 0))],
        out_specs=[pl.BlockSpec((_NG, nout), lambda i: (0, 0)),
                   pl.BlockSpec((_NG, 1), lambda i: (0, 0))],
        out_shape=[jax.ShapeDtypeStruct((_NG, nout), jnp.float32),
                   jax.ShapeDtypeStruct((_NG, 1), jnp.float32)],
    )(ma, mb, s1, f1, bt3, wp1, wp2, wp3, wsc, wr, br)


def kernel(atomic_numbers, edge_attr, edge_index, batch, W_embed, R1, R2, R3,
           R4, W_up, W_sc, W_species, Wp1, Wp2, Wp3, W_readout, b_readout):
    n = atomic_numbers.shape[0]
    e = edge_attr.shape[0]
    src2 = edge_index[0].astype(jnp.int32).reshape(e // _CH, _CH)
    dst2 = edge_index[1].astype(jnp.int32).reshape(e // _CH, _CH)
    an3 = atomic_numbers.astype(jnp.int32).reshape(n // _NB, 1, _NB)
    bt3 = batch.astype(jnp.int32).reshape(n // _NB, 1, _NB)

    f0up, spec0, spec1 = _node_prep(an3, W_embed, W_up[0],
                                    W_species[0], W_species[1])
    w0 = _edge_mlp(edge_attr, R1[0], R2[0], R3[0], R4[0][:, :_HID])
    w1 = _edge_mlp(edge_attr, R1[1], R2[1], R3[1], R4[1][:, :_HID])
    m0a, m0b = _sc_msg(f0up, w0, src2, dst2)
    f1, f1up = _layer0_update(m0a, m0b, spec0, Wp1[0], Wp2[0], Wp3[0],
                              W_up[1])
    m1a, m1b = _sc_msg(f1up, w1, src2, dst2)
    sums, counts = _final(m1a, m1b, spec1, f1, bt3, Wp1[1], Wp2[1], Wp3[1],
                          W_sc, W_readout, b_readout.reshape(1, -1))
    return sums / jnp.maximum(counts, 1.0)


# speedup vs baseline: 108.8408x; 1.1743x over previous
import functools

import jax
import jax.numpy as jnp
from jax import lax
from jax.experimental import pallas as pl
from jax.experimental.pallas import tpu as pltpu
from jax.experimental.pallas import tpu_sc as plsc

_HID = 128
_NB = 2000
_EB = 2000
_CH = 128
_NC, _NS = 2, 16
_NW = _NC * _NS
_NG = 64
_AVG = 35.0
_RMAX = 5.0


def _silu(x):
    return x * (1.0 / (1.0 + jnp.exp(-x)))


def _dot(a, b):
    return jnp.dot(a, b, preferred_element_type=jnp.float32)


def _dotT(aT, b):
    return lax.dot_general(aT, b, (((0,), (0,)), ((), ())),
                           preferred_element_type=jnp.float32)


def _edge_mlp_body(ea_ref, r1_ref, r2_ref, r3_ref, r40_ref, r41_ref,
                   w0_ref, w1_ref):
    ea = ea_ref[...]
    r = jnp.sqrt(jnp.sum(ea * ea, axis=1, keepdims=True))
    nfreq = lax.broadcasted_iota(jnp.int32, (1, 8), 1).astype(jnp.float32) + 1.0
    pref = jnp.sqrt(2.0 / _RMAX)
    bess = pref * jnp.sin(nfreq * (jnp.pi / _RMAX) * r) / jnp.maximum(r, 1e-6)
    u = r * (1.0 / _RMAX)
    u2 = u * u
    u5 = u2 * u2 * u
    env = 1.0 - 21.0 * u5 + 35.0 * u5 * u - 15.0 * u5 * u2
    env = jnp.where(u < 1.0, env, 0.0)
    ef = bess * env
    h = _silu(_dot(ef, r1_ref[...]))
    h = _silu(_dot(h, r2_ref[...]))
    h = _silu(_dot(h, r3_ref[...]))
    w0_ref[...] = _dot(h[:, :64], r40_ref[...])
    w1_ref[...] = _dot(h[:, 64:], r41_ref[...])


def _edge_mlp2(edge_attr, r1c, r2c, r3c, r40, r41):
    e = edge_attr.shape[0]
    grid = (e // _EB,)
    out = jax.ShapeDtypeStruct((e, _HID), jnp.float32)
    return pl.pallas_call(
        _edge_mlp_body,
        grid=grid,
        in_specs=[
            pl.BlockSpec((_EB, 3), lambda i: (i, 0)),
            pl.BlockSpec(r1c.shape, lambda i: (0, 0)),
            pl.BlockSpec(r2c.shape, lambda i: (0, 0)),
            pl.BlockSpec(r3c.shape, lambda i: (0, 0)),
            pl.BlockSpec(r40.shape, lambda i: (0, 0)),
            pl.BlockSpec(r41.shape, lambda i: (0, 0)),
        ],
        out_specs=[pl.BlockSpec((_EB, _HID), lambda i: (i, 0))] * 2,
        out_shape=[out, out],
    )(edge_attr, r1c, r2c, r3c, r40, r41)


def _node_prep_body(an_ref, we_ref, wu_ref, ws0_ref, ws1_ref,
                    f0up_ref, s0_ref, s1_ref):
    an = an_ref[0]
    ids = lax.broadcasted_iota(jnp.int32, (we_ref.shape[0], 1), 0)
    oht = (ids == an).astype(jnp.float32)
    z0 = _dotT(oht, we_ref[...])
    f0up_ref[...] = _dot(z0, wu_ref[...])
    s0_ref[...] = _dotT(oht, ws0_ref[...])
    s1_ref[...] = _dotT(oht, ws1_ref[...])


def _node_prep(an3, w_embed, w_up0, ws0, ws1):
    nblk = an3.shape[0]
    n = nblk * _NB
    out = jax.ShapeDtypeStruct((n, _HID), jnp.float32)
    return pl.pallas_call(
        _node_prep_body,
        grid=(nblk,),
        in_specs=[
            pl.BlockSpec((1, 1, _NB), lambda i: (i, 0, 0)),
            pl.BlockSpec(w_embed.shape, lambda i: (0, 0)),
            pl.BlockSpec(w_up0.shape, lambda i: (0, 0)),
            pl.BlockSpec(ws0.shape, lambda i: (0, 0)),
            pl.BlockSpec(ws1.shape, lambda i: (0, 0)),
        ],
        out_specs=[pl.BlockSpec((_NB, _HID), lambda i: (i, 0))] * 3,
        out_shape=[out] * 3,
    )(an3, w_embed, w_up0, ws0, ws1)


def _sc_msg(f_nodes, w_edges, src2, dst2):
    n, d = f_nodes.shape
    n_chunks = src2.shape[0]
    per_w = n_chunks // _NW
    extra = n_chunks - per_w * _NW
    rbase = 624
    rlast = n - rbase * (_NS - 1)
    mesh = plsc.VectorSubcoreMesh(core_axis_name="c", subcore_axis_name="s")
    out_t = [jax.ShapeDtypeStruct((n, d), jnp.float32)] * 2

    @functools.partial(
        pl.kernel, out_type=out_t, mesh=mesh,
        scratch_types=[
            pltpu.VMEM((1, _CH), jnp.int32),
            pltpu.VMEM((1, _CH), jnp.int32),
            pltpu.VMEM((_CH, d), jnp.float32),
            pltpu.VMEM((_CH, d), jnp.float32),
            pltpu.VMEM_SHARED((n, d), jnp.float32),
            pltpu.SemaphoreType.DMA,
        ])
    def k(f_hbm, w_hbm, src_hbm, dst_hbm, out0, out1,
          src_v, dst_v, se_v, w_v, acc, sem):
        cid = lax.axis_index("c")
        sid = lax.axis_index("s")
        wid = cid * _NS + sid

        base = sid * rbase

        @pl.loop(0, _CH)
        def _z(rr):
            for cc in range(d // 16):
                w_v[rr, pl.ds(cc * 16, 16)] = jnp.zeros((16,), jnp.float32)

        @pl.loop(0, rbase // _CH)
        def _zc(j):
            pltpu.sync_copy(w_v, acc.at[pl.ds(base + j * _CH, _CH)])

        @pl.when(sid < _NS - 1)
        def _zt():
            pltpu.sync_copy(w_v.at[pl.ds(0, rbase % _CH)],
                            acc.at[pl.ds(base + rbase - rbase % _CH,
                                         rbase % _CH)])

        @pl.when(sid == _NS - 1)
        def _zt2():
            pltpu.sync_copy(w_v, acc.at[pl.ds(base + (rbase // _CH) * _CH,
                                              rlast - (rbase // _CH) * _CH)])

        plsc.subcore_barrier()

        nb = per_w + jnp.where(wid < extra, 1, 0)

        @pl.loop(0, per_w + 1)
        def _c(c):
            @pl.when(c < nb)
            def _():
                g = wid + c * _NW
                pltpu.sync_copy(src_hbm.at[pl.ds(g, 1)], src_v)
                pltpu.sync_copy(dst_hbm.at[pl.ds(g, 1)], dst_v)
                pltpu.async_copy(f_hbm.at[src_v.at[0]], se_v, sem).wait()
                pltpu.sync_copy(w_hbm.at[pl.ds(g * _CH, _CH)], w_v)

                @pl.loop(0, _CH)
                def _m(rr):
                    for cc in range(d // 16):
                        sl = pl.ds(cc * 16, 16)
                        w_v[rr, sl] = w_v[rr, sl] * se_v[rr, sl]

                pltpu.sync_copy(w_v, acc.at[dst_v.at[0]], add=True)

        plsc.subcore_barrier()

        @pl.when(cid == 0)
        def _o0():
            @pl.when(sid < _NS - 1)
            def _():
                pltpu.sync_copy(acc.at[pl.ds(base, rbase)],
                                out0.at[pl.ds(base, rbase)])

            @pl.when(sid == _NS - 1)
            def _():
                pltpu.sync_copy(acc.at[pl.ds(base, rlast)],
                                out0.at[pl.ds(base, rlast)])

        @pl.when(cid == 1)
        def _o1():
            @pl.when(sid < _NS - 1)
            def _():
                pltpu.sync_copy(acc.at[pl.ds(base, rbase)],
                                out1.at[pl.ds(base, rbase)])

            @pl.when(sid == _NS - 1)
            def _():
                pltpu.sync_copy(acc.at[pl.ds(base, rlast)],
                                out1.at[pl.ds(base, rlast)])

    return k(f_nodes, w_edges, src2, dst2)


def _l0_body(ma_ref, mb_ref, s0_ref, wp1_ref, wp2_ref, wp3_ref, wu1_ref,
             f1_ref, f1up_ref):
    msg = (ma_ref[...] + mb_ref[...]) * (1.0 / _AVG)
    msg2 = msg * msg
    o = _dot(msg, wp1_ref[...])
    o = o + _dot(msg2, wp2_ref[...])
    o = o + _dot(msg2 * msg, wp3_ref[...])
    f1 = o * s0_ref[...]
    f1_ref[...] = f1
    f1up_ref[...] = _dot(f1, wu1_ref[...])


def _layer0_update(ma, mb, s0, wp1, wp2, wp3, wu1):
    n = ma.shape[0]
    out = jax.ShapeDtypeStruct((n, _HID), jnp.float32)
    full = pl.BlockSpec((_HID, _HID), lambda i: (0, 0))
    blk = pl.BlockSpec((_NB, _HID), lambda i: (i, 0))
    return pl.pallas_call(
        _l0_body,
        grid=(n // _NB,),
        in_specs=[blk, blk, blk, full, full, full, full],
        out_specs=[blk, blk],
        out_shape=[out, out],
    )(ma, mb, s0, wp1, wp2, wp3, wu1)


def _final_body(ma_ref, mb_ref, s1_ref, f1_ref, bt_ref, wp1_ref, wp2_ref,
                wp3_ref, wsc_ref, wr_ref, br_ref, sums_ref, cnt_ref):
    i = pl.program_id(0)
    msg = (ma_ref[...] + mb_ref[...]) * (1.0 / _AVG)
    msg2 = msg * msg
    o = _dot(msg, wp1_ref[...])
    o = o + _dot(msg2, wp2_ref[...])
    o = o + _dot(msg2 * msg, wp3_ref[...])
    o = o * s1_ref[...]
    o = o + _dot(f1_ref[...], wsc_ref[...])
    nout = _dot(o, wr_ref[...]) + br_ref[...]
    bt = bt_ref[0]
    gids = lax.broadcasted_iota(jnp.int32, (_NG, 1), 0)
    oht = (gids == bt).astype(jnp.float32)
    s = _dot(oht, nout)
    c = jnp.sum(oht, axis=1, keepdims=True)

    @pl.when(i == 0)
    def _():
        sums_ref[...] = jnp.zeros_like(sums_ref)
        cnt_ref[...] = jnp.zeros_like(cnt_ref)

    sums_ref[...] += s
    cnt_ref[...] += c


def _final(ma, mb, s1, f1, bt3, wp1, wp2, wp3, wsc, wr, br):
    n = ma.shape[0]
    nout = wr.shape[1]
    blk = pl.BlockSpec((_NB, _HID), lambda i: (i, 0))
    full = pl.BlockSpec((_HID, _HID), lambda i: (0, 0))
    return pl.pallas_call(
        _final_body,
        grid=(n // _NB,),
        in_specs=[blk, blk, blk, blk,
                  pl.BlockSpec((1, 1, _NB), lambda i: (i, 0, 0)),
                  full, full, full, full,
                  pl.BlockSpec(wr.shape, lambda i: (0, 0)),
                  pl.BlockSpec(br.shape, lambda i: (0, 0))],
        out_specs=[pl.BlockSpec((_NG, nout), lambda i: (0, 0)),
                   pl.BlockSpec((_NG, 1), lambda i: (0, 0))],
        out_shape=[jax.ShapeDtypeStruct((_NG, nout), jnp.float32),
                   jax.ShapeDtypeStruct((_NG, 1), jnp.float32)],
    )(ma, mb, s1, f1, bt3, wp1, wp2, wp3, wsc, wr, br)


def kernel(atomic_numbers, edge_attr, edge_index, batch, W_embed, R1, R2, R3,
           R4, W_up, W_sc, W_species, Wp1, Wp2, Wp3, W_readout, b_readout):
    n = atomic_numbers.shape[0]
    e = edge_attr.shape[0]
    src2 = edge_index[0].astype(jnp.int32).reshape(e // _CH, _CH)
    dst2 = edge_index[1].astype(jnp.int32).reshape(e // _CH, _CH)
    an3 = atomic_numbers.astype(jnp.int32).reshape(n // _NB, 1, _NB)
    bt3 = batch.astype(jnp.int32).reshape(n // _NB, 1, _NB)

    r1c = jnp.concatenate([R1[0], R1[1]], axis=1)
    z64 = jnp.zeros((64, 64), jnp.float32)
    r2c = jnp.block([[R2[0], z64], [z64, R2[1]]])
    r3c = jnp.block([[R3[0], z64], [z64, R3[1]]])

    f0up, spec0, spec1 = _node_prep(an3, W_embed, W_up[0],
                                    W_species[0], W_species[1])
    w0, w1 = _edge_mlp2(edge_attr, r1c, r2c, r3c,
                        R4[0][:, :_HID], R4[1][:, :_HID])
    m0a, m0b = _sc_msg(f0up, w0, src2, dst2)
    f1, f1up = _layer0_update(m0a, m0b, spec0, Wp1[0], Wp2[0], Wp3[0],
                              W_up[1])
    m1a, m1b = _sc_msg(f1up, w1, src2, dst2)
    sums, counts = _final(m1a, m1b, spec1, f1, bt3, Wp1[1], Wp2[1], Wp3[1],
                          W_sc, W_readout, b_readout.reshape(1, -1))
    return sums / jnp.maximum(counts, 1.0)


# speedup vs baseline: 168.0805x; 1.5443x over previous
import functools

import jax
import jax.numpy as jnp
from jax import lax
from jax.experimental import pallas as pl
from jax.experimental.pallas import tpu as pltpu
from jax.experimental.pallas import tpu_sc as plsc

_HID = 128
_NB = 2000
_EB = 6400
_CH = 128
_NC, _NS = 2, 16
_NW = _NC * _NS
_NG = 64
_AVG = 35.0
_RMAX = 5.0


def _silu(x):
    return x * (1.0 / (1.0 + jnp.exp(-x)))


def _dot(a, b):
    return jnp.dot(a, b, preferred_element_type=jnp.float32)


def _dotT(aT, b):
    return lax.dot_general(aT, b, (((0,), (0,)), ((), ())),
                           preferred_element_type=jnp.float32)


def _edge_mlp_body(eaT_ref, r1_ref, r2_ref, r3_ref, r40_ref,
                   w0_ref, h1_ref):
    ea = eaT_ref[...]
    r = jnp.sqrt(jnp.sum(ea * ea, axis=0, keepdims=True))
    nfreq = lax.broadcasted_iota(jnp.int32, (8, 1), 0).astype(jnp.float32) + 1.0
    pref = jnp.sqrt(2.0 / _RMAX)
    bess = pref * jnp.sin(nfreq * (jnp.pi / _RMAX) * r) / jnp.maximum(r, 1e-6)
    u = r * (1.0 / _RMAX)
    u2 = u * u
    u5 = u2 * u2 * u
    env = 1.0 - 21.0 * u5 + 35.0 * u5 * u - 15.0 * u5 * u2
    env = jnp.where(u < 1.0, env, 0.0)
    ef = bess * env
    h = _silu(_dotT(ef, r1_ref[...]))
    h = _silu(_dot(h, r2_ref[...]))
    h = _silu(_dot(h, r3_ref[...]))
    w0_ref[...] = _dot(h[:, :64], r40_ref[...])
    h1_ref[...] = h[:, 64:]


def _edge_mlp2(edge_attrT, r1c, r2c, r3c, r40):
    e = edge_attrT.shape[1]
    grid = (e // _EB,)
    return pl.pallas_call(
        _edge_mlp_body,
        grid=grid,
        in_specs=[
            pl.BlockSpec((3, _EB), lambda i: (0, i)),
            pl.BlockSpec(r1c.shape, lambda i: (0, 0)),
            pl.BlockSpec(r2c.shape, lambda i: (0, 0)),
            pl.BlockSpec(r3c.shape, lambda i: (0, 0)),
            pl.BlockSpec(r40.shape, lambda i: (0, 0)),
        ],
        out_specs=[pl.BlockSpec((_EB, _HID), lambda i: (i, 0)),
                   pl.BlockSpec((_EB, 64), lambda i: (i, 0))],
        out_shape=[jax.ShapeDtypeStruct((e, _HID), jnp.float32),
                   jax.ShapeDtypeStruct((e, 64), jnp.float32)],
    )(edge_attrT, r1c, r2c, r3c, r40)


def _w1_body(h1_ref, r41_ref, w1_ref):
    w1_ref[...] = _dot(h1_ref[...], r41_ref[...])


def _w1_mlp(h1, r41):
    e = h1.shape[0]
    return pl.pallas_call(
        _w1_body,
        grid=(e // _EB,),
        in_specs=[pl.BlockSpec((_EB, 64), lambda i: (i, 0)),
                  pl.BlockSpec(r41.shape, lambda i: (0, 0))],
        out_specs=pl.BlockSpec((_EB, _HID), lambda i: (i, 0)),
        out_shape=jax.ShapeDtypeStruct((e, _HID), jnp.float32),
    )(h1, r41)


def _node_prep_body(an_ref, we_ref, wu_ref, ws0_ref, ws1_ref,
                    f0up_ref, s0_ref, s1_ref):
    an = an_ref[0]
    ids = lax.broadcasted_iota(jnp.int32, (we_ref.shape[0], 1), 0)
    oht = (ids == an).astype(jnp.float32)
    z0 = _dotT(oht, we_ref[...])
    f0up_ref[...] = _dot(z0, wu_ref[...])
    s0_ref[...] = _dotT(oht, ws0_ref[...])
    s1_ref[...] = _dotT(oht, ws1_ref[...])


def _node_prep(an3, w_embed, w_up0, ws0, ws1):
    nblk = an3.shape[0]
    n = nblk * _NB
    out = jax.ShapeDtypeStruct((n, _HID), jnp.float32)
    return pl.pallas_call(
        _node_prep_body,
        grid=(nblk,),
        in_specs=[
            pl.BlockSpec((1, 1, _NB), lambda i: (i, 0, 0)),
            pl.BlockSpec(w_embed.shape, lambda i: (0, 0)),
            pl.BlockSpec(w_up0.shape, lambda i: (0, 0)),
            pl.BlockSpec(ws0.shape, lambda i: (0, 0)),
            pl.BlockSpec(ws1.shape, lambda i: (0, 0)),
        ],
        out_specs=[pl.BlockSpec((_NB, _HID), lambda i: (i, 0))] * 3,
        out_shape=[out] * 3,
    )(an3, w_embed, w_up0, ws0, ws1)


def _sc_msg(f_nodes, w_edges, src2, dst2):
    n, d = f_nodes.shape
    n_chunks = src2.shape[0]
    per_w = n_chunks // _NW
    extra = n_chunks - per_w * _NW
    rbase = 624
    rlast = n - rbase * (_NS - 1)
    mesh = plsc.VectorSubcoreMesh(core_axis_name="c", subcore_axis_name="s")
    out_t = [jax.ShapeDtypeStruct((n, d), jnp.float32)] * 2

    @functools.partial(
        pl.kernel, out_type=out_t, mesh=mesh,
        scratch_types=[
            pltpu.VMEM((1, _CH), jnp.int32),
            pltpu.VMEM((1, _CH), jnp.int32),
            pltpu.VMEM((_CH, d), jnp.float32),
            pltpu.VMEM((_CH, d), jnp.float32),
            pltpu.VMEM_SHARED((n, d), jnp.float32),
            pltpu.SemaphoreType.DMA,
        ])
    def k(f_hbm, w_hbm, src_hbm, dst_hbm, out0, out1,
          src_v, dst_v, se_v, w_v, acc, sem):
        cid = lax.axis_index("c")
        sid = lax.axis_index("s")
        wid = cid * _NS + sid

        base = sid * rbase

        @pl.loop(0, _CH)
        def _z(rr):
            for cc in range(d // 16):
                w_v[rr, pl.ds(cc * 16, 16)] = jnp.zeros((16,), jnp.float32)

        @pl.loop(0, rbase // _CH)
        def _zc(j):
            pltpu.sync_copy(w_v, acc.at[pl.ds(base + j * _CH, _CH)])

        @pl.when(sid < _NS - 1)
        def _zt():
            pltpu.sync_copy(w_v.at[pl.ds(0, rbase % _CH)],
                            acc.at[pl.ds(base + rbase - rbase % _CH,
                                         rbase % _CH)])

        @pl.when(sid == _NS - 1)
        def _zt2():
            pltpu.sync_copy(w_v, acc.at[pl.ds(base + (rbase // _CH) * _CH,
                                              rlast - (rbase // _CH) * _CH)])

        plsc.subcore_barrier()

        nb = per_w + jnp.where(wid < extra, 1, 0)

        @pl.loop(0, per_w + 1)
        def _c(c):
            @pl.when(c < nb)
            def _():
                g = wid + c * _NW
                pltpu.sync_copy(src_hbm.at[pl.ds(g, 1)], src_v)
                pltpu.sync_copy(dst_hbm.at[pl.ds(g, 1)], dst_v)
                pltpu.async_copy(f_hbm.at[src_v.at[0]], se_v, sem).wait()
                pltpu.sync_copy(w_hbm.at[pl.ds(g * _CH, _CH)], w_v)

                @pl.loop(0, _CH)
                def _m(rr):
                    for cc in range(d // 16):
                        sl = pl.ds(cc * 16, 16)
                        w_v[rr, sl] = w_v[rr, sl] * se_v[rr, sl]

                pltpu.sync_copy(w_v, acc.at[dst_v.at[0]], add=True)

        plsc.subcore_barrier()

        @pl.when(cid == 0)
        def _o0():
            @pl.when(sid < _NS - 1)
            def _():
                pltpu.sync_copy(acc.at[pl.ds(base, rbase)],
                                out0.at[pl.ds(base, rbase)])

            @pl.when(sid == _NS - 1)
            def _():
                pltpu.sync_copy(acc.at[pl.ds(base, rlast)],
                                out0.at[pl.ds(base, rlast)])

        @pl.when(cid == 1)
        def _o1():
            @pl.when(sid < _NS - 1)
            def _():
                pltpu.sync_copy(acc.at[pl.ds(base, rbase)],
                                out1.at[pl.ds(base, rbase)])

            @pl.when(sid == _NS - 1)
            def _():
                pltpu.sync_copy(acc.at[pl.ds(base, rlast)],
                                out1.at[pl.ds(base, rlast)])

    return k(f_nodes, w_edges, src2, dst2)


def _l0_body(ma_ref, mb_ref, s0_ref, wp1_ref, wp2_ref, wp3_ref, wu1_ref,
             f1_ref, f1up_ref):
    msg = (ma_ref[...] + mb_ref[...]) * (1.0 / _AVG)
    msg2 = msg * msg
    o = _dot(msg, wp1_ref[...])
    o = o + _dot(msg2, wp2_ref[...])
    o = o + _dot(msg2 * msg, wp3_ref[...])
    f1 = o * s0_ref[...]
    f1_ref[...] = f1
    f1up_ref[...] = _dot(f1, wu1_ref[...])


def _layer0_update(ma, mb, s0, wp1, wp2, wp3, wu1):
    n = ma.shape[0]
    out = jax.ShapeDtypeStruct((n, _HID), jnp.float32)
    full = pl.BlockSpec((_HID, _HID), lambda i: (0, 0))
    blk = pl.BlockSpec((_NB, _HID), lambda i: (i, 0))
    return pl.pallas_call(
        _l0_body,
        grid=(n // _NB,),
        in_specs=[blk, blk, blk, full, full, full, full],
        out_specs=[blk, blk],
        out_shape=[out, out],
    )(ma, mb, s0, wp1, wp2, wp3, wu1)


def _final_body(ma_ref, mb_ref, s1_ref, f1_ref, bt_ref, wp1_ref, wp2_ref,
                wp3_ref, wsc_ref, wr_ref, br_ref, sums_ref, cnt_ref):
    i = pl.program_id(0)
    msg = (ma_ref[...] + mb_ref[...]) * (1.0 / _AVG)
    msg2 = msg * msg
    o = _dot(msg, wp1_ref[...])
    o = o + _dot(msg2, wp2_ref[...])
    o = o + _dot(msg2 * msg, wp3_ref[...])
    o = o * s1_ref[...]
    o = o + _dot(f1_ref[...], wsc_ref[...])
    nout = _dot(o, wr_ref[...]) + br_ref[...]
    bt = bt_ref[0]
    gids = lax.broadcasted_iota(jnp.int32, (_NG, 1), 0)
    oht = (gids == bt).astype(jnp.float32)
    s = _dot(oht, nout)
    c = jnp.sum(oht, axis=1, keepdims=True)

    @pl.when(i == 0)
    def _():
        sums_ref[...] = jnp.zeros_like(sums_ref)
        cnt_ref[...] = jnp.zeros_like(cnt_ref)

    sums_ref[...] += s
    cnt_ref[...] += c


def _final(ma, mb, s1, f1, bt3, wp1, wp2, wp3, wsc, wr, br):
    n = ma.shape[0]
    nout = wr.shape[1]
    blk = pl.BlockSpec((_NB, _HID), lambda i: (i, 0))
    full = pl.BlockSpec((_HID, _HID), lambda i: (0, 0))
    return pl.pallas_call(
        _final_body,
        grid=(n // _NB,),
        in_specs=[blk, blk, blk, blk,
                  pl.BlockSpec((1, 1, _NB), lambda i: (i, 0, 0)),
                  full, full, full, full,
                  pl.BlockSpec(wr.shape, lambda i: (0, 0)),
                  pl.BlockSpec(br.shape, lambda i: (0, 0))],
        out_specs=[pl.BlockSpec((_NG, nout), lambda i: (0, 0)),
                   pl.BlockSpec((_NG, 1), lambda i: (0, 0))],
        out_shape=[jax.ShapeDtypeStruct((_NG, nout), jnp.float32),
                   jax.ShapeDtypeStruct((_NG, 1), jnp.float32)],
    )(ma, mb, s1, f1, bt3, wp1, wp2, wp3, wsc, wr, br)


def kernel(atomic_numbers, edge_attr, edge_index, batch, W_embed, R1, R2, R3,
           R4, W_up, W_sc, W_species, Wp1, Wp2, Wp3, W_readout, b_readout):
    n = atomic_numbers.shape[0]
    e = edge_attr.shape[0]
    src2 = edge_index[0].astype(jnp.int32).reshape(e // _CH, _CH)
    dst2 = edge_index[1].astype(jnp.int32).reshape(e // _CH, _CH)
    an3 = atomic_numbers.astype(jnp.int32).reshape(n // _NB, 1, _NB)
    bt3 = batch.astype(jnp.int32).reshape(n // _NB, 1, _NB)

    r1c = jnp.concatenate([R1[0], R1[1]], axis=1)
    z64 = jnp.zeros((64, 64), jnp.float32)
    r2c = jnp.block([[R2[0], z64], [z64, R2[1]]])
    r3c = jnp.block([[R3[0], z64], [z64, R3[1]]])

    f0up, spec0, spec1 = _node_prep(an3, W_embed, W_up[0],
                                    W_species[0], W_species[1])
    w0, h1 = _edge_mlp2(edge_attr.T, r1c, r2c, r3c, R4[0][:, :_HID])
    m0a, m0b = _sc_msg(f0up, w0, src2, dst2)
    w1 = _w1_mlp(h1, R4[1][:, :_HID])
    f1, f1up = _layer0_update(m0a, m0b, spec0, Wp1[0], Wp2[0], Wp3[0],
                              W_up[1])
    m1a, m1b = _sc_msg(f1up, w1, src2, dst2)
    sums, counts = _final(m1a, m1b, spec1, f1, bt3, Wp1[1], Wp2[1], Wp3[1],
                          W_sc, W_readout, b_readout.reshape(1, -1))
    return sums / jnp.maximum(counts, 1.0)


# speedup vs baseline: 248.6946x; 1.4796x over previous
import functools

import jax
import jax.numpy as jnp
from jax import lax
from jax.experimental import pallas as pl
from jax.experimental.pallas import tpu as pltpu
from jax.experimental.pallas import tpu_sc as plsc

_HID = 128
_NB = 2000
_EB = 6400
_CH = 80
_NC, _NS = 2, 16
_NW = _NC * _NS
_NG = 64
_AVG = 35.0
_RMAX = 5.0


def _silu(x):
    return x * (1.0 / (1.0 + jnp.exp(-x)))


def _dot(a, b):
    return jnp.dot(a, b, preferred_element_type=jnp.float32)


def _dotT(aT, b):
    return lax.dot_general(aT, b, (((0,), (0,)), ((), ())),
                           preferred_element_type=jnp.float32)


def _edge_mlp_body(eaT_ref, r1_ref, r2_ref, r3_ref, r40_ref,
                   w0_ref, h1_ref):
    ea = eaT_ref[...]
    r = jnp.sqrt(jnp.sum(ea * ea, axis=0, keepdims=True))
    nfreq = lax.broadcasted_iota(jnp.int32, (8, 1), 0).astype(jnp.float32) + 1.0
    pref = jnp.sqrt(2.0 / _RMAX)
    bess = pref * jnp.sin(nfreq * (jnp.pi / _RMAX) * r) / jnp.maximum(r, 1e-6)
    u = r * (1.0 / _RMAX)
    u2 = u * u
    u5 = u2 * u2 * u
    env = 1.0 - 21.0 * u5 + 35.0 * u5 * u - 15.0 * u5 * u2
    env = jnp.where(u < 1.0, env, 0.0)
    ef = bess * env
    h = _silu(_dotT(ef, r1_ref[...]))
    h = _silu(_dot(h, r2_ref[...]))
    h = _silu(_dot(h, r3_ref[...]))
    w0_ref[...] = _dot(h[:, :64], r40_ref[...])
    h1_ref[...] = h[:, 64:]


def _edge_mlp2(edge_attrT, r1c, r2c, r3c, r40):
    e = edge_attrT.shape[1]
    grid = (e // _EB,)
    return pl.pallas_call(
        _edge_mlp_body,
        grid=grid,
        in_specs=[
            pl.BlockSpec((3, _EB), lambda i: (0, i)),
            pl.BlockSpec(r1c.shape, lambda i: (0, 0)),
            pl.BlockSpec(r2c.shape, lambda i: (0, 0)),
            pl.BlockSpec(r3c.shape, lambda i: (0, 0)),
            pl.BlockSpec(r40.shape, lambda i: (0, 0)),
        ],
        out_specs=[pl.BlockSpec((_EB, _HID), lambda i: (i, 0)),
                   pl.BlockSpec((_EB, 64), lambda i: (i, 0))],
        out_shape=[jax.ShapeDtypeStruct((e, _HID), jnp.float32),
                   jax.ShapeDtypeStruct((e, 64), jnp.float32)],
    )(edge_attrT, r1c, r2c, r3c, r40)


def _w1_body(h1_ref, r41_ref, w1_ref):
    w1_ref[...] = _dot(h1_ref[...], r41_ref[...])


def _w1_mlp(h1, r41):
    e = h1.shape[0]
    return pl.pallas_call(
        _w1_body,
        grid=(e // _EB,),
        in_specs=[pl.BlockSpec((_EB, 64), lambda i: (i, 0)),
                  pl.BlockSpec(r41.shape, lambda i: (0, 0))],
        out_specs=pl.BlockSpec((_EB, _HID), lambda i: (i, 0)),
        out_shape=jax.ShapeDtypeStruct((e, _HID), jnp.float32),
    )(h1, r41)


def _node_prep_body(an_ref, we_ref, wu_ref, ws0_ref, ws1_ref,
                    f0up_ref, s0_ref, s1_ref):
    an = an_ref[0]
    ids = lax.broadcasted_iota(jnp.int32, (we_ref.shape[0], 1), 0)
    oht = (ids == an).astype(jnp.float32)
    z0 = _dotT(oht, we_ref[...])
    f0up_ref[...] = _dot(z0, wu_ref[...])
    s0_ref[...] = _dotT(oht, ws0_ref[...])
    s1_ref[...] = _dotT(oht, ws1_ref[...])


def _node_prep(an3, w_embed, w_up0, ws0, ws1):
    nblk = an3.shape[0]
    n = nblk * _NB
    out = jax.ShapeDtypeStruct((n, _HID), jnp.float32)
    return pl.pallas_call(
        _node_prep_body,
        grid=(nblk,),
        in_specs=[
            pl.BlockSpec((1, 1, _NB), lambda i: (i, 0, 0)),
            pl.BlockSpec(w_embed.shape, lambda i: (0, 0)),
            pl.BlockSpec(w_up0.shape, lambda i: (0, 0)),
            pl.BlockSpec(ws0.shape, lambda i: (0, 0)),
            pl.BlockSpec(ws1.shape, lambda i: (0, 0)),
        ],
        out_specs=[pl.BlockSpec((_NB, _HID), lambda i: (i, 0))] * 3,
        out_shape=[out] * 3,
    )(an3, w_embed, w_up0, ws0, ws1)


def _sc_msg(f_nodes, w_edges, ei3):
    n, d = f_nodes.shape
    n_chunks = ei3.shape[0]
    per_w = n_chunks // _NW
    extra = n_chunks - per_w * _NW
    nmax = -(-(per_w + 1) // 2) * 2
    rbase = 624
    rlast = n - rbase * (_NS - 1)
    mesh = plsc.VectorSubcoreMesh(core_axis_name="c", subcore_axis_name="s")
    out_t = [jax.ShapeDtypeStruct((n, d), jnp.float32)] * 2

    @functools.partial(
        pl.kernel, out_type=out_t, mesh=mesh,
        scratch_types=[
            pltpu.VMEM((2, _CH), jnp.int32),
            pltpu.VMEM((2, _CH), jnp.int32),
            pltpu.VMEM((_CH, d), jnp.float32),
            pltpu.VMEM((_CH, d), jnp.float32),
            pltpu.VMEM((_CH, d), jnp.float32),
            pltpu.VMEM((_CH, d), jnp.float32),
            pltpu.VMEM_SHARED((n, d), jnp.float32),
            pltpu.SemaphoreType.DMA,
            pltpu.SemaphoreType.DMA,
            pltpu.SemaphoreType.DMA,
            pltpu.SemaphoreType.DMA,
        ])
    def k(f_hbm, w_hbm, ei_hbm, out0, out1,
          i0, i1, se0, se1, w0, w1, acc, sg0, sg1, sw0, sw1):
        cid = lax.axis_index("c")
        sid = lax.axis_index("s")
        wid = cid * _NS + sid
        bufs = ((i0, se0, w0, sg0, sw0), (i1, se1, w1, sg1, sw1))

        base = sid * rbase

        @pl.loop(0, _CH)
        def _z(rr):
            for cc in range(d // 16):
                w0[rr, pl.ds(cc * 16, 16)] = jnp.zeros((16,), jnp.float32)

        @pl.loop(0, rbase // _CH)
        def _zc(j):
            pltpu.sync_copy(w0, acc.at[pl.ds(base + j * _CH, _CH)])

        @pl.when(sid < _NS - 1)
        def _zt():
            pltpu.sync_copy(w0.at[pl.ds(0, rbase % _CH)],
                            acc.at[pl.ds(base + rbase - rbase % _CH,
                                         rbase % _CH)])

        @pl.when(sid == _NS - 1)
        def _zt2():
            pltpu.sync_copy(w0, acc.at[pl.ds(base + (rbase // _CH) * _CH,
                                             rlast - (rbase // _CH) * _CH)])

        plsc.subcore_barrier()

        nb = per_w + jnp.where(wid < extra, 1, 0)

        @pl.when(nb > 0)
        def _prime():
            pltpu.sync_copy(ei_hbm.at[wid], i0)
            pltpu.async_copy(f_hbm.at[i0.at[0]], se0, sg0)
            pltpu.async_copy(w_hbm.at[pl.ds(wid * _CH, _CH)], w0, sw0)

        @pl.loop(0, nmax, step=2)
        def _c(j):
            for b in (0, 1):
                c = j + b
                ib, seb, wb, sgb, swb = bufs[b]
                io, seo, wo, sgo, swo = bufs[1 - b]

                @pl.when(c + 1 < nb)
                def _pref():
                    g1 = wid + (c + 1) * _NW
                    pltpu.sync_copy(ei_hbm.at[g1], io)
                    pltpu.async_copy(f_hbm.at[io.at[0]], seo, sgo)
                    pltpu.async_copy(w_hbm.at[pl.ds(g1 * _CH, _CH)], wo, swo)

                @pl.when(c < nb)
                def _work():
                    pltpu.make_async_copy(f_hbm.at[pl.ds(0, _CH)], seb,
                                          sgb).wait()
                    pltpu.make_async_copy(f_hbm.at[pl.ds(0, _CH)], wb,
                                          swb).wait()

                    @pl.loop(0, _CH, step=4)
                    def _m(rr):
                        for dr in range(4):
                            for cc in range(d // 16):
                                sl = pl.ds(cc * 16, 16)
                                wb[rr + dr, sl] = (wb[rr + dr, sl]
                                                   * seb[rr + dr, sl])

                    pltpu.sync_copy(wb, acc.at[ib.at[1]], add=True)

        plsc.subcore_barrier()

        @pl.when(cid == 0)
        def _o0():
            @pl.when(sid < _NS - 1)
            def _():
                pltpu.sync_copy(acc.at[pl.ds(base, rbase)],
                                out0.at[pl.ds(base, rbase)])

            @pl.when(sid == _NS - 1)
            def _():
                pltpu.sync_copy(acc.at[pl.ds(base, rlast)],
                                out0.at[pl.ds(base, rlast)])

        @pl.when(cid == 1)
        def _o1():
            @pl.when(sid < _NS - 1)
            def _():
                pltpu.sync_copy(acc.at[pl.ds(base, rbase)],
                                out1.at[pl.ds(base, rbase)])

            @pl.when(sid == _NS - 1)
            def _():
                pltpu.sync_copy(acc.at[pl.ds(base, rlast)],
                                out1.at[pl.ds(base, rlast)])

    return k(f_nodes, w_edges, ei3)


def _l0_body(ma_ref, mb_ref, s0_ref, wp1_ref, wp2_ref, wp3_ref, wu1_ref,
             f1_ref, f1up_ref):
    msg = (ma_ref[...] + mb_ref[...]) * (1.0 / _AVG)
    msg2 = msg * msg
    o = _dot(msg, wp1_ref[...])
    o = o + _dot(msg2, wp2_ref[...])
    o = o + _dot(msg2 * msg, wp3_ref[...])
    f1 = o * s0_ref[...]
    f1_ref[...] = f1
    f1up_ref[...] = _dot(f1, wu1_ref[...])


def _layer0_update(ma, mb, s0, wp1, wp2, wp3, wu1):
    n = ma.shape[0]
    out = jax.ShapeDtypeStruct((n, _HID), jnp.float32)
    full = pl.BlockSpec((_HID, _HID), lambda i: (0, 0))
    blk = pl.BlockSpec((_NB, _HID), lambda i: (i, 0))
    return pl.pallas_call(
        _l0_body,
        grid=(n // _NB,),
        in_specs=[blk, blk, blk, full, full, full, full],
        out_specs=[blk, blk],
        out_shape=[out, out],
    )(ma, mb, s0, wp1, wp2, wp3, wu1)


def _final_body(ma_ref, mb_ref, s1_ref, f1_ref, bt_ref, wp1_ref, wp2_ref,
                wp3_ref, wsc_ref, wr_ref, br_ref, sums_ref, cnt_ref):
    i = pl.program_id(0)
    msg = (ma_ref[...] + mb_ref[...]) * (1.0 / _AVG)
    msg2 = msg * msg
    o = _dot(msg, wp1_ref[...])
    o = o + _dot(msg2, wp2_ref[...])
    o = o + _dot(msg2 * msg, wp3_ref[...])
    o = o * s1_ref[...]
    o = o + _dot(f1_ref[...], wsc_ref[...])
    nout = _dot(o, wr_ref[...]) + br_ref[...]
    bt = bt_ref[0]
    gids = lax.broadcasted_iota(jnp.int32, (_NG, 1), 0)
    oht = (gids == bt).astype(jnp.float32)
    s = _dot(oht, nout)
    c = jnp.sum(oht, axis=1, keepdims=True)

    @pl.when(i == 0)
    def _():
        sums_ref[...] = jnp.zeros_like(sums_ref)
        cnt_ref[...] = jnp.zeros_like(cnt_ref)

    sums_ref[...] += s
    cnt_ref[...] += c


def _final(ma, mb, s1, f1, bt3, wp1, wp2, wp3, wsc, wr, br):
    n = ma.shape[0]
    nout = wr.shape[1]
    blk = pl.BlockSpec((_NB, _HID), lambda i: (i, 0))
    full = pl.BlockSpec((_HID, _HID), lambda i: (0, 0))
    return pl.pallas_call(
        _final_body,
        grid=(n // _NB,),
        in_specs=[blk, blk, blk, blk,
                  pl.BlockSpec((1, 1, _NB), lambda i: (i, 0, 0)),
                  full, full, full, full,
                  pl.BlockSpec(wr.shape, lambda i: (0, 0)),
                  pl.BlockSpec(br.shape, lambda i: (0, 0))],
        out_specs=[pl.BlockSpec((_NG, nout), lambda i: (0, 0)),
                   pl.BlockSpec((_NG, 1), lambda i: (0, 0))],
        out_shape=[jax.ShapeDtypeStruct((_NG, nout), jnp.float32),
                   jax.ShapeDtypeStruct((_NG, 1), jnp.float32)],
    )(ma, mb, s1, f1, bt3, wp1, wp2, wp3, wsc, wr, br)


def kernel(atomic_numbers, edge_attr, edge_index, batch, W_embed, R1, R2, R3,
           R4, W_up, W_sc, W_species, Wp1, Wp2, Wp3, W_readout, b_readout):
    n = atomic_numbers.shape[0]
    e = edge_attr.shape[0]
    ei3 = edge_index.astype(jnp.int32).reshape(2, e // _CH, _CH)
    ei3 = jnp.swapaxes(ei3, 0, 1)
    an3 = atomic_numbers.astype(jnp.int32).reshape(n // _NB, 1, _NB)
    bt3 = batch.astype(jnp.int32).reshape(n // _NB, 1, _NB)

    r1c = jnp.concatenate([R1[0], R1[1]], axis=1)
    z64 = jnp.zeros((64, 64), jnp.float32)
    r2c = jnp.block([[R2[0], z64], [z64, R2[1]]])
    r3c = jnp.block([[R3[0], z64], [z64, R3[1]]])

    f0up, spec0, spec1 = _node_prep(an3, W_embed, W_up[0],
                                    W_species[0], W_species[1])
    w0, h1 = _edge_mlp2(edge_attr.T, r1c, r2c, r3c, R4[0][:, :_HID])
    m0a, m0b = _sc_msg(f0up, w0, ei3)
    w1 = _w1_mlp(h1, R4[1][:, :_HID])
    f1, f1up = _layer0_update(m0a, m0b, spec0, Wp1[0], Wp2[0], Wp3[0],
                              W_up[1])
    m1a, m1b = _sc_msg(f1up, w1, ei3)
    sums, counts = _final(m1a, m1b, spec1, f1, bt3, Wp1[1], Wp2[1], Wp3[1],
                          W_sc, W_readout, b_readout.reshape(1, -1))
    return sums / jnp.maximum(counts, 1.0)


# speedup vs baseline: 251.6411x; 1.0118x over previous
import functools

import jax
import jax.numpy as jnp
from jax import lax
from jax.experimental import pallas as pl
from jax.experimental.pallas import tpu as pltpu
from jax.experimental.pallas import tpu_sc as plsc

_HID = 128
_NB = 2000
_EB = 6400
_CH = 80
_NC, _NS = 2, 16
_NW = _NC * _NS
_NG = 64
_AVG = 35.0
_RMAX = 5.0


def _silu(x):
    return x * (1.0 / (1.0 + jnp.exp(-x)))


def _dot(a, b):
    return jnp.dot(a, b, preferred_element_type=jnp.float32)


def _dotT(aT, b):
    return lax.dot_general(aT, b, (((0,), (0,)), ((), ())),
                           preferred_element_type=jnp.float32)


def _edge_mlp_body(eaT_ref, r1_ref, r2_ref, r3_ref, r40_ref,
                   w0_ref, h1_ref):
    ea = eaT_ref[...]
    r = jnp.sqrt(jnp.sum(ea * ea, axis=0, keepdims=True))
    nfreq = lax.broadcasted_iota(jnp.int32, (8, 1), 0).astype(jnp.float32) + 1.0
    pref = jnp.sqrt(2.0 / _RMAX)
    bess = pref * jnp.sin(nfreq * (jnp.pi / _RMAX) * r) / jnp.maximum(r, 1e-6)
    u = r * (1.0 / _RMAX)
    u2 = u * u
    u5 = u2 * u2 * u
    env = 1.0 - 21.0 * u5 + 35.0 * u5 * u - 15.0 * u5 * u2
    env = jnp.where(u < 1.0, env, 0.0)
    ef = bess * env
    h = _silu(_dotT(ef, r1_ref[...]))
    h = _silu(_dot(h.astype(jnp.bfloat16), r2_ref[...]))
    h = _silu(_dot(h.astype(jnp.bfloat16), r3_ref[...]))
    hb = h.astype(jnp.bfloat16)
    w0_ref[...] = _dot(hb[:, :64], r40_ref[...])
    h1_ref[...] = hb[:, 64:]


def _edge_mlp2(edge_attrT, r1c, r2c, r3c, r40):
    e = edge_attrT.shape[1]
    grid = (e // _EB,)
    return pl.pallas_call(
        _edge_mlp_body,
        grid=grid,
        in_specs=[
            pl.BlockSpec((3, _EB), lambda i: (0, i)),
            pl.BlockSpec(r1c.shape, lambda i: (0, 0)),
            pl.BlockSpec(r2c.shape, lambda i: (0, 0)),
            pl.BlockSpec(r3c.shape, lambda i: (0, 0)),
            pl.BlockSpec(r40.shape, lambda i: (0, 0)),
        ],
        out_specs=[pl.BlockSpec((_EB, _HID), lambda i: (i, 0)),
                   pl.BlockSpec((_EB, 64), lambda i: (i, 0))],
        out_shape=[jax.ShapeDtypeStruct((e, _HID), jnp.float32),
                   jax.ShapeDtypeStruct((e, 64), jnp.bfloat16)],
    )(edge_attrT, r1c, r2c, r3c, r40)


def _w1_body(h1_ref, r41_ref, w1_ref):
    w1_ref[...] = _dot(h1_ref[...], r41_ref[...])


def _w1_mlp(h1, r41):
    e = h1.shape[0]
    return pl.pallas_call(
        _w1_body,
        grid=(e // _EB,),
        in_specs=[pl.BlockSpec((_EB, 64), lambda i: (i, 0)),
                  pl.BlockSpec(r41.shape, lambda i: (0, 0))],
        out_specs=pl.BlockSpec((_EB, _HID), lambda i: (i, 0)),
        out_shape=jax.ShapeDtypeStruct((e, _HID), jnp.float32),
    )(h1, r41)


def _node_prep_body(an_ref, we_ref, wu_ref, ws0_ref, ws1_ref,
                    f0up_ref, s0_ref, s1_ref):
    an = an_ref[0]
    ids = lax.broadcasted_iota(jnp.int32, (we_ref.shape[0], 1), 0)
    oht = (ids == an).astype(jnp.float32)
    z0 = _dotT(oht, we_ref[...])
    f0up_ref[...] = _dot(z0, wu_ref[...])
    s0_ref[...] = _dotT(oht, ws0_ref[...])
    s1_ref[...] = _dotT(oht, ws1_ref[...])


def _node_prep(an3, w_embed, w_up0, ws0, ws1):
    nblk = an3.shape[0]
    n = nblk * _NB
    out = jax.ShapeDtypeStruct((n, _HID), jnp.float32)
    return pl.pallas_call(
        _node_prep_body,
        grid=(nblk,),
        in_specs=[
            pl.BlockSpec((1, 1, _NB), lambda i: (i, 0, 0)),
            pl.BlockSpec(w_embed.shape, lambda i: (0, 0)),
            pl.BlockSpec(w_up0.shape, lambda i: (0, 0)),
            pl.BlockSpec(ws0.shape, lambda i: (0, 0)),
            pl.BlockSpec(ws1.shape, lambda i: (0, 0)),
        ],
        out_specs=[pl.BlockSpec((_NB, _HID), lambda i: (i, 0))] * 3,
        out_shape=[out] * 3,
    )(an3, w_embed, w_up0, ws0, ws1)


def _sc_msg(f_nodes, w_edges, ei3):
    n, d = f_nodes.shape
    n_chunks = ei3.shape[0]
    per_w = n_chunks // _NW
    extra = n_chunks - per_w * _NW
    nmax = -(-(per_w + 1) // 2) * 2
    rbase = 624
    rlast = n - rbase * (_NS - 1)
    mesh = plsc.VectorSubcoreMesh(core_axis_name="c", subcore_axis_name="s")
    out_t = [jax.ShapeDtypeStruct((n, d), jnp.float32)] * 2

    @functools.partial(
        pl.kernel, out_type=out_t, mesh=mesh,
        scratch_types=[
            pltpu.VMEM((2, _CH), jnp.int32),
            pltpu.VMEM((2, _CH), jnp.int32),
            pltpu.VMEM((_CH, d), jnp.float32),
            pltpu.VMEM((_CH, d), jnp.float32),
            pltpu.VMEM((_CH, d), jnp.float32),
            pltpu.VMEM((_CH, d), jnp.float32),
            pltpu.VMEM_SHARED((n, d), jnp.float32),
            pltpu.SemaphoreType.DMA,
            pltpu.SemaphoreType.DMA,
            pltpu.SemaphoreType.DMA,
            pltpu.SemaphoreType.DMA,
        ])
    def k(f_hbm, w_hbm, ei_hbm, out0, out1,
          i0, i1, se0, se1, w0, w1, acc, sg0, sg1, sw0, sw1):
        cid = lax.axis_index("c")
        sid = lax.axis_index("s")
        wid = cid * _NS + sid
        bufs = ((i0, se0, w0, sg0, sw0), (i1, se1, w1, sg1, sw1))

        base = sid * rbase

        @pl.loop(0, _CH)
        def _z(rr):
            for cc in range(d // 16):
                w0[rr, pl.ds(cc * 16, 16)] = jnp.zeros((16,), jnp.float32)

        @pl.loop(0, rbase // _CH)
        def _zc(j):
            pltpu.sync_copy(w0, acc.at[pl.ds(base + j * _CH, _CH)])

        @pl.when(sid < _NS - 1)
        def _zt():
            pltpu.sync_copy(w0.at[pl.ds(0, rbase % _CH)],
                            acc.at[pl.ds(base + rbase - rbase % _CH,
                                         rbase % _CH)])

        @pl.when(sid == _NS - 1)
        def _zt2():
            pltpu.sync_copy(w0, acc.at[pl.ds(base + (rbase // _CH) * _CH,
                                             rlast - (rbase // _CH) * _CH)])

        plsc.subcore_barrier()

        nb = per_w + jnp.where(wid < extra, 1, 0)

        @pl.when(nb > 0)
        def _prime():
            pltpu.sync_copy(ei_hbm.at[wid], i0)
            pltpu.async_copy(f_hbm.at[i0.at[0]], se0, sg0)
            pltpu.async_copy(w_hbm.at[pl.ds(wid * _CH, _CH)], w0, sw0)

        @pl.loop(0, nmax, step=2)
        def _c(j):
            for b in (0, 1):
                c = j + b
                ib, seb, wb, sgb, swb = bufs[b]
                io, seo, wo, sgo, swo = bufs[1 - b]

                @pl.when(c + 1 < nb)
                def _pref():
                    g1 = wid + (c + 1) * _NW
                    pltpu.sync_copy(ei_hbm.at[g1], io)
                    pltpu.async_copy(f_hbm.at[io.at[0]], seo, sgo)
                    pltpu.async_copy(w_hbm.at[pl.ds(g1 * _CH, _CH)], wo, swo)

                @pl.when(c < nb)
                def _work():
                    pltpu.make_async_copy(f_hbm.at[pl.ds(0, _CH)], seb,
                                          sgb).wait()
                    pltpu.make_async_copy(f_hbm.at[pl.ds(0, _CH)], wb,
                                          swb).wait()

                    @pl.loop(0, _CH, step=4)
                    def _m(rr):
                        for dr in range(4):
                            for cc in range(d // 16):
                                sl = pl.ds(cc * 16, 16)
                                wb[rr + dr, sl] = (wb[rr + dr, sl]
                                                   * seb[rr + dr, sl])

                    pltpu.sync_copy(wb, acc.at[ib.at[1]], add=True)

        plsc.subcore_barrier()

        @pl.when(cid == 0)
        def _o0():
            @pl.when(sid < _NS - 1)
            def _():
                pltpu.sync_copy(acc.at[pl.ds(base, rbase)],
                                out0.at[pl.ds(base, rbase)])

            @pl.when(sid == _NS - 1)
            def _():
                pltpu.sync_copy(acc.at[pl.ds(base, rlast)],
                                out0.at[pl.ds(base, rlast)])

        @pl.when(cid == 1)
        def _o1():
            @pl.when(sid < _NS - 1)
            def _():
                pltpu.sync_copy(acc.at[pl.ds(base, rbase)],
                                out1.at[pl.ds(base, rbase)])

            @pl.when(sid == _NS - 1)
            def _():
                pltpu.sync_copy(acc.at[pl.ds(base, rlast)],
                                out1.at[pl.ds(base, rlast)])

    return k(f_nodes, w_edges, ei3)


def _l0_body(ma_ref, mb_ref, s0_ref, wp1_ref, wp2_ref, wp3_ref, wu1_ref,
             f1_ref, f1up_ref):
    msg = (ma_ref[...] + mb_ref[...]) * (1.0 / _AVG)
    msg2 = msg * msg
    o = _dot(msg, wp1_ref[...])
    o = o + _dot(msg2, wp2_ref[...])
    o = o + _dot(msg2 * msg, wp3_ref[...])
    f1 = o * s0_ref[...]
    f1_ref[...] = f1
    f1up_ref[...] = _dot(f1, wu1_ref[...])


def _layer0_update(ma, mb, s0, wp1, wp2, wp3, wu1):
    n = ma.shape[0]
    out = jax.ShapeDtypeStruct((n, _HID), jnp.float32)
    full = pl.BlockSpec((_HID, _HID), lambda i: (0, 0))
    blk = pl.BlockSpec((_NB, _HID), lambda i: (i, 0))
    return pl.pallas_call(
        _l0_body,
        grid=(n // _NB,),
        in_specs=[blk, blk, blk, full, full, full, full],
        out_specs=[blk, blk],
        out_shape=[out, out],
    )(ma, mb, s0, wp1, wp2, wp3, wu1)


def _final_body(ma_ref, mb_ref, s1_ref, f1_ref, bt_ref, wp1_ref, wp2_ref,
                wp3_ref, wsc_ref, wr_ref, br_ref, sums_ref, cnt_ref):
    i = pl.program_id(0)
    msg = (ma_ref[...] + mb_ref[...]) * (1.0 / _AVG)
    msg2 = msg * msg
    o = _dot(msg, wp1_ref[...])
    o = o + _dot(msg2, wp2_ref[...])
    o = o + _dot(msg2 * msg, wp3_ref[...])
    o = o * s1_ref[...]
    o = o + _dot(f1_ref[...], wsc_ref[...])
    nout = _dot(o, wr_ref[...]) + br_ref[...]
    bt = bt_ref[0]
    gids = lax.broadcasted_iota(jnp.int32, (_NG, 1), 0)
    oht = (gids == bt).astype(jnp.float32)
    s = _dot(oht, nout)
    c = jnp.sum(oht, axis=1, keepdims=True)

    @pl.when(i == 0)
    def _():
        sums_ref[...] = jnp.zeros_like(sums_ref)
        cnt_ref[...] = jnp.zeros_like(cnt_ref)

    sums_ref[...] += s
    cnt_ref[...] += c


def _final(ma, mb, s1, f1, bt3, wp1, wp2, wp3, wsc, wr, br):
    n = ma.shape[0]
    nout = wr.shape[1]
    blk = pl.BlockSpec((_NB, _HID), lambda i: (i, 0))
    full = pl.BlockSpec((_HID, _HID), lambda i: (0, 0))
    return pl.pallas_call(
        _final_body,
        grid=(n // _NB,),
        in_specs=[blk, blk, blk, blk,
                  pl.BlockSpec((1, 1, _NB), lambda i: (i, 0, 0)),
                  full, full, full, full,
                  pl.BlockSpec(wr.shape, lambda i: (0, 0)),
                  pl.BlockSpec(br.shape, lambda i: (0, 0))],
        out_specs=[pl.BlockSpec((_NG, nout), lambda i: (0, 0)),
                   pl.BlockSpec((_NG, 1), lambda i: (0, 0))],
        out_shape=[jax.ShapeDtypeStruct((_NG, nout), jnp.float32),
                   jax.ShapeDtypeStruct((_NG, 1), jnp.float32)],
    )(ma, mb, s1, f1, bt3, wp1, wp2, wp3, wsc, wr, br)


def kernel(atomic_numbers, edge_attr, edge_index, batch, W_embed, R1, R2, R3,
           R4, W_up, W_sc, W_species, Wp1, Wp2, Wp3, W_readout, b_readout):
    n = atomic_numbers.shape[0]
    e = edge_attr.shape[0]
    ei3 = edge_index.astype(jnp.int32).reshape(2, e // _CH, _CH)
    ei3 = jnp.swapaxes(ei3, 0, 1)
    an3 = atomic_numbers.astype(jnp.int32).reshape(n // _NB, 1, _NB)
    bt3 = batch.astype(jnp.int32).reshape(n // _NB, 1, _NB)

    r1c = jnp.concatenate([R1[0], R1[1]], axis=1)
    z64 = jnp.zeros((64, 64), jnp.float32)
    r2c = jnp.block([[R2[0], z64], [z64, R2[1]]]).astype(jnp.bfloat16)
    r3c = jnp.block([[R3[0], z64], [z64, R3[1]]]).astype(jnp.bfloat16)

    f0up, spec0, spec1 = _node_prep(an3, W_embed, W_up[0],
                                    W_species[0], W_species[1])
    w0, h1 = _edge_mlp2(edge_attr.T, r1c, r2c, r3c,
                        R4[0][:, :_HID].astype(jnp.bfloat16))
    m0a, m0b = _sc_msg(f0up, w0, ei3)
    w1 = _w1_mlp(h1, R4[1][:, :_HID].astype(jnp.bfloat16))
    f1, f1up = _layer0_update(m0a, m0b, spec0, Wp1[0], Wp2[0], Wp3[0],
                              W_up[1])
    m1a, m1b = _sc_msg(f1up, w1, ei3)
    sums, counts = _final(m1a, m1b, spec1, f1, bt3, Wp1[1], Wp2[1], Wp3[1],
                          W_sc, W_readout, b_readout.reshape(1, -1))
    return sums / jnp.maximum(counts, 1.0)
